# baseline (speedup 1.0000x reference)
import functools

import jax
import jax.numpy as jnp
from jax import lax
from jax.experimental import pallas as pl
from jax.experimental.pallas import tpu as pltpu

B, N = 4, 100000
IN_DIM, FEAT = 4, 128
H, W = 128, 128
PCR = [-50.0, -50.0, -5.0, 50.0, 50.0, 3.0]
NTOT = B * N
BLK = 3200
NSTEP = NTOT // BLK
EPS = 1e-5


def _affine(sums_row, sumsq_row, gamma, beta):
    mean = sums_row / NTOT
    var = sumsq_row / NTOT - mean * mean
    inv = lax.rsqrt(var + EPS)
    scale = gamma * inv
    shift = beta - mean * scale
    return scale, shift


def _layer1(pts, W1T_ref, b1_ref):
    h = b1_ref[...].reshape(1, 64)
    for c in range(IN_DIM):
        h = h + pts[:, c:c + 1] * W1T_ref[c:c + 1, :]
    return h


def _dot(a, w_ref):
    return lax.dot_general(a, w_ref[...], (((1,), (0,)), ((), ())),
                           precision=lax.Precision.HIGHEST,
                           preferred_element_type=jnp.float32)


def _accum_stats(ref, h, step):
    s = jnp.sum(h, axis=0, keepdims=True)
    ss = jnp.sum(h * h, axis=0, keepdims=True)
    blockstat = jnp.concatenate([s, ss], axis=0)

    @pl.when(step == 0)
    def _():
        ref[...] = blockstat

    @pl.when(step != 0)
    def _():
        ref[...] += blockstat


def _k1_body(pts_ref, W1T_ref, b1_ref, sums1_ref, flat_ref):
    step = pl.program_id(0)
    pts = pts_ref[...]
    h1 = _layer1(pts, W1T_ref, b1_ref)
    _accum_stats(sums1_ref, h1, step)
    xn = (pts[:, 0:1] - PCR[0]) / (PCR[3] - PCR[0])
    yn = (pts[:, 1:2] - PCR[1]) / (PCR[4] - PCR[1])
    gx = jnp.clip((xn * (W - 1)).astype(jnp.int32), 0, W - 1)
    gy = jnp.clip((yn * (H - 1)).astype(jnp.int32), 0, H - 1)
    gidx = step * BLK + lax.broadcasted_iota(jnp.int32, (BLK, 1), 0)
    b = gidx // N
    flat_ref[...] = b * (H * W) + gy * W + gx


def _k2_body(pts_ref, W1T_ref, b1_ref, g1_ref, be1_ref, W2T_ref, b2_ref,
             sums1_ref, sums2_ref):
    step = pl.program_id(0)
    pts = pts_ref[...]
    h1 = _layer1(pts, W1T_ref, b1_ref)
    sc1, sh1 = _affine(sums1_ref[0:1, :], sums1_ref[1:2, :], g1_ref[...], be1_ref[...])
    a1 = jnp.maximum(h1 * sc1 + sh1, 0.0)
    h2 = _dot(a1, W2T_ref) + b2_ref[...].reshape(1, FEAT)
    _accum_stats(sums2_ref, h2, step)


def _k3_body(pts_ref, W1T_ref, b1_ref, g1_ref, be1_ref, W2T_ref, b2_ref,
             g2_ref, be2_ref, W3T_ref, b3_ref, sums1_ref, sums2_ref,
             sums3_ref):
    step = pl.program_id(0)
    pts = pts_ref[...]
    h1 = _layer1(pts, W1T_ref, b1_ref)
    sc1, sh1 = _affine(sums1_ref[0:1, :], sums1_ref[1:2, :], g1_ref[...], be1_ref[...])
    a1 = jnp.maximum(h1 * sc1 + sh1, 0.0)
    h2 = _dot(a1, W2T_ref) + b2_ref[...].reshape(1, FEAT)
    sc2, sh2 = _affine(sums2_ref[0:1, :], sums2_ref[1:2, :], g2_ref[...], be2_ref[...])
    a2 = jnp.maximum(h2 * sc2 + sh2, 0.0)
    h3 = _dot(a2, W3T_ref) + b3_ref[...].reshape(1, FEAT)
    _accum_stats(sums3_ref, h3, step)


def _k4_body(flat_ref, pts_ref, W1T_ref, b1_ref, g1_ref, be1_ref, W2T_ref,
             b2_ref, g2_ref, be2_ref, W3T_ref, b3_ref, g3_ref, be3_ref,
             sums1_ref, sums2_ref, sums3_ref, grid_ref, feats_ref):
    step = pl.program_id(0)
    pts = pts_ref[...]
    h1 = _layer1(pts, W1T_ref, b1_ref)
    sc1, sh1 = _affine(sums1_ref[0:1, :], sums1_ref[1:2, :], g1_ref[...], be1_ref[...])
    a1 = jnp.maximum(h1 * sc1 + sh1, 0.0)
    h2 = _dot(a1, W2T_ref) + b2_ref[...].reshape(1, FEAT)
    sc2, sh2 = _affine(sums2_ref[0:1, :], sums2_ref[1:2, :], g2_ref[...], be2_ref[...])
    a2 = jnp.maximum(h2 * sc2 + sh2, 0.0)
    h3 = _dot(a2, W3T_ref) + b3_ref[...].reshape(1, FEAT)
    sc3, sh3 = _affine(sums3_ref[0:1, :], sums3_ref[1:2, :], g3_ref[...], be3_ref[...])
    a3 = jnp.maximum(h3 * sc3 + sh3, 0.0)

    xn = (pts[:, 0:1] - PCR[0]) / (PCR[3] - PCR[0])
    yn = (pts[:, 1:2] - PCR[1]) / (PCR[4] - PCR[1])
    valid = (xn >= 0) & (xn <= 1) & (yn >= 0) & (yn <= 1)
    feats_ref[...] = jnp.where(valid, a3, 0.0)

    @pl.when(step == 0)
    def _():
        grid_ref[...] = jnp.zeros_like(grid_ref)

    def body(i, _):
        idx = flat_ref[0, 0, i]
        row = grid_ref[pl.ds(idx, 1), :]
        grid_ref[pl.ds(idx, 1), :] = jnp.maximum(row, feats_ref[pl.ds(i, 1), :])
        return 0

    lax.fori_loop(0, BLK, body, 0, unroll=False)


def kernel(points, W1, b1, g1, be1, W2, b2, g2, be2, W3, b3, g3, be3):
    pts = points.reshape(NTOT, IN_DIM)
    W1T, W2T, W3T = W1.T, W2.T, W3.T

    pspec = pl.BlockSpec((BLK, IN_DIM), lambda i: (i, 0))
    full = pl.BlockSpec(None, lambda i: tuple(0 for _ in range(2)))

    def wspec(arr):
        return pl.BlockSpec(arr.shape, lambda i: tuple(0 for _ in arr.shape))

    statspec = pl.BlockSpec((2, None), lambda i: (0, 0))

    sums1, flat = pl.pallas_call(
        _k1_body,
        grid=(NSTEP,),
        in_specs=[pspec, wspec(W1T), wspec(b1)],
        out_specs=[pl.BlockSpec((2, 64), lambda i: (0, 0)),
                   pl.BlockSpec((BLK, 1), lambda i: (i, 0))],
        out_shape=[jax.ShapeDtypeStruct((2, 64), jnp.float32),
                   jax.ShapeDtypeStruct((NTOT, 1), jnp.int32)],
    )(pts, W1T, b1)

    sums2 = pl.pallas_call(
        _k2_body,
        grid=(NSTEP,),
        in_specs=[pspec, wspec(W1T), wspec(b1), wspec(g1), wspec(be1),
                  wspec(W2T), wspec(b2), pl.BlockSpec((2, 64), lambda i: (0, 0))],
        out_specs=pl.BlockSpec((2, FEAT), lambda i: (0, 0)),
        out_shape=jax.ShapeDtypeStruct((2, FEAT), jnp.float32),
    )(pts, W1T, b1, g1, be1, W2T, b2, sums1)

    sums3 = pl.pallas_call(
        _k3_body,
        grid=(NSTEP,),
        in_specs=[pspec, wspec(W1T), wspec(b1), wspec(g1), wspec(be1),
                  wspec(W2T), wspec(b2), wspec(g2), wspec(be2),
                  wspec(W3T), wspec(b3),
                  pl.BlockSpec((2, 64), lambda i: (0, 0)),
                  pl.BlockSpec((2, FEAT), lambda i: (0, 0))],
        out_specs=pl.BlockSpec((2, FEAT), lambda i: (0, 0)),
        out_shape=jax.ShapeDtypeStruct((2, FEAT), jnp.float32),
    )(pts, W1T, b1, g1, be1, W2T, b2, g2, be2, W3T, b3, sums1, sums2)

    flat3 = flat.reshape(NSTEP, 1, BLK)
    grid_out = pl.pallas_call(
        _k4_body,
        grid=(NSTEP,),
        in_specs=[pl.BlockSpec((1, 1, BLK), lambda i: (i, 0, 0),
                               memory_space=pltpu.SMEM),
                  pspec, wspec(W1T), wspec(b1), wspec(g1), wspec(be1),
                  wspec(W2T), wspec(b2), wspec(g2), wspec(be2),
                  wspec(W3T), wspec(b3), wspec(g3), wspec(be3),
                  pl.BlockSpec((2, 64), lambda i: (0, 0)),
                  pl.BlockSpec((2, FEAT), lambda i: (0, 0)),
                  pl.BlockSpec((2, FEAT), lambda i: (0, 0))],
        out_specs=pl.BlockSpec((B * H * W, FEAT), lambda i: (0, 0)),
        out_shape=jax.ShapeDtypeStruct((B * H * W, FEAT), jnp.float32),
        scratch_shapes=[pltpu.VMEM((BLK, FEAT), jnp.float32)],
    )(flat3, pts, W1T, b1, g1, be1, W2T, b2, g2, be2, W3T, b3, g3, be3,
      sums1, sums2, sums3)

    fm = grid_out.reshape(B, H, W, FEAT)
    return jnp.transpose(fm, (0, 3, 1, 2))
